# baseline (speedup 1.0000x reference)
import functools

import jax
import jax.numpy as jnp
from jax import lax
from jax.experimental import pallas as pl
from jax.experimental.pallas import tpu as pltpu
from jax.experimental.pallas import tpu_sc as plsc

T = 2048
D = 1024
E = 8
K = 2
FF = 768
SFF = 2048

BT = 256
NT = 24
PADT = NT * BT

NC = 2
NS = 16
NW = NC * NS


def _router_body(x_ref, wg_ref, pos_ref, wexp_ref, te_ref, nr_ref):
    x = x_ref[...]
    logits = lax.dot_general(wg_ref[...], x, (((1,), (1,)), ((), ())),
                             preferred_element_type=jnp.float32)
    m = jnp.max(logits, axis=0, keepdims=True)
    ex = jnp.exp(logits - m)
    probs = ex / jnp.sum(ex, axis=0, keepdims=True)
    eids = lax.broadcasted_iota(jnp.int32, (E, T), 0)
    m1 = jnp.max(probs, axis=0, keepdims=True)
    i1 = jnp.min(jnp.where(probs == m1, eids, E), axis=0, keepdims=True)
    mask1 = eids == i1
    probs2 = jnp.where(mask1, -1.0, probs)
    m2 = jnp.max(probs2, axis=0, keepdims=True)
    i2 = jnp.min(jnp.where(probs2 == m2, eids, E), axis=0, keepdims=True)
    mask2 = eids == i2
    s = m1 + m2
    w1 = m1 / s
    w2 = m2 / s

    A = mask1.astype(jnp.float32) + mask2.astype(jnp.float32)
    r_io = lax.broadcasted_iota(jnp.int32, (128, 128), 0)
    c_io = lax.broadcasted_iota(jnp.int32, (128, 128), 1)
    LT = (r_io < c_io).astype(jnp.float32)
    carry = jnp.zeros((E, 1), jnp.float32)
    chunks = []
    for c in range(T // 128):
        Xc = A[:, c * 128:(c + 1) * 128]
        Sc = lax.dot_general(Xc, LT, (((1,), (0,)), ((), ())),
                             preferred_element_type=jnp.float32) + carry
        carry = carry + jnp.sum(Xc, axis=1, keepdims=True)
        chunks.append(Sc)
    S = jnp.concatenate(chunks, axis=1)
    counts = carry
    padded = jnp.ceil(counts / BT) * BT
    tr = lax.broadcasted_iota(jnp.int32, (E, E), 0)
    tc_ = lax.broadcasted_iota(jnp.int32, (E, E), 1)
    tri = (tc_ < tr).astype(jnp.float32)
    base = lax.dot_general(tri, padded, (((1,), (0,)), ((), ())),
                           preferred_element_type=jnp.float32)
    ptotal = jnp.sum(padded)

    P = base + S
    pos0 = jnp.sum(jnp.where(mask1, P, 0.0), axis=0, keepdims=True)
    pos1 = jnp.sum(jnp.where(mask2, P, 0.0), axis=0, keepdims=True)
    pos_ref[...] = jnp.concatenate([pos0, pos1], axis=1).astype(jnp.int32).reshape(K * T)
    w1t = jnp.transpose(w1, (1, 0))
    w2t = jnp.transpose(w2, (1, 0))
    wexp_ref[...] = jnp.concatenate(
        [jnp.broadcast_to(w1t, (T, 16)), jnp.broadcast_to(w2t, (T, 16))], axis=1)

    tl = lax.broadcasted_iota(jnp.int32, (1, 128), 1).astype(jnp.float32) * BT
    belongs = (tl >= base) & (tl < base + padded)
    eids_p = lax.broadcasted_iota(jnp.int32, (E, 128), 0)
    tile_e = jnp.sum(jnp.where(belongs, eids_p, 0), axis=0, keepdims=True)
    tile_e = jnp.where(tl < ptotal, tile_e, E - 1)
    te_ref[...] = tile_e.reshape(128)
    nreal = (ptotal / BT).astype(jnp.int32)
    nr_ref[...] = jnp.zeros((8,), jnp.int32) + nreal


def _router(x, Wg):
    return pl.pallas_call(
        _router_body,
        out_shape=(
            jax.ShapeDtypeStruct((K * T,), jnp.int32),
            jax.ShapeDtypeStruct((T, K * 16), jnp.float32),
            jax.ShapeDtypeStruct((128,), jnp.int32),
            jax.ShapeDtypeStruct((8,), jnp.int32),
        ),
    )(x, Wg)


def _scatter_sc(x, pos_flat):
    mesh = plsc.VectorSubcoreMesh(core_axis_name="c", subcore_axis_name="s")
    tok_per_w = T // NW
    ch = 32
    nch = tok_per_w // ch

    @functools.partial(
        pl.kernel, mesh=mesh,
        out_type=jax.ShapeDtypeStruct((PADT, D), jnp.float32),
        scratch_types=[
            pltpu.VMEM((ch,), jnp.int32), pltpu.VMEM((ch,), jnp.int32),
            pltpu.VMEM((ch,), jnp.int32), pltpu.VMEM((ch,), jnp.int32),
            pltpu.VMEM((ch, D), jnp.float32), pltpu.VMEM((ch, D), jnp.float32),
            pltpu.SemaphoreType.DMA, pltpu.SemaphoreType.DMA,
            pltpu.SemaphoreType.DMA, pltpu.SemaphoreType.DMA,
        ],
    )
    def k(x_hbm, pos_hbm, xs_out,
          i0a, i0b, i1a, i1b, rows0, rows1, ls0, ls1, ss0, ss1):
        wid = lax.axis_index("s") * NC + lax.axis_index("c")
        i0_v = (i0a, i0b)
        i1_v = (i1a, i1b)
        rows_v = (rows0, rows1)
        lsem = (ls0, ls1)
        ssem = (ss0, ss1)
        lh = {}
        sh = {}

        def fire_loads(c):
            b = c % 2
            base = wid * tok_per_w + c * ch
            lh[c] = (
                pltpu.async_copy(pos_hbm.at[pl.ds(base, ch)], i0_v[b], lsem[b]),
                pltpu.async_copy(pos_hbm.at[pl.ds(T + base, ch)], i1_v[b], lsem[b]),
                pltpu.async_copy(x_hbm.at[pl.ds(base, ch)], rows_v[b], lsem[b]),
            )

        for c in range(nch):
            fire_loads(c)
        for c in range(nch):
            b = c % 2
            for h in lh[c]:
                h.wait()
            sh[c] = (
                pltpu.async_copy(rows_v[b], xs_out.at[i0_v[b]], ssem[b]),
                pltpu.async_copy(rows_v[b], xs_out.at[i1_v[b]], ssem[b]),
            )
        for c in range(nch):
            for h in sh[c]:
                h.wait()

    return k(x, pos_flat)


def _group_body(s0_ref, s1_ref, xs_ref, wg_ref, wu_ref, wd_ref, out_ref):
    t = pl.program_id(0)

    @pl.when(t < s1_ref[0])
    def _():
        x = xs_ref[...]
        g = lax.dot_general(x, wg_ref[0], (((1,), (1,)), ((), ())),
                            preferred_element_type=jnp.float32)
        u = lax.dot_general(x, wu_ref[0], (((1,), (1,)), ((), ())),
                            preferred_element_type=jnp.float32)
        a = g * jax.nn.sigmoid(g) * u
        out_ref[...] = lax.dot_general(a, wd_ref[0], (((1,), (1,)), ((), ())),
                                       preferred_element_type=jnp.float32)


def _group_mm(tile_e, nreal, Xs, We_gate, We_up, We_down):
    grid_spec = pltpu.PrefetchScalarGridSpec(
        num_scalar_prefetch=2,
        grid=(NT,),
        in_specs=[
            pl.BlockSpec((BT, D), lambda t, s0, s1: (t, 0)),
            pl.BlockSpec((1, FF, D), lambda t, s0, s1: (s0[t], 0, 0)),
            pl.BlockSpec((1, FF, D), lambda t, s0, s1: (s0[t], 0, 0)),
            pl.BlockSpec((1, D, FF), lambda t, s0, s1: (s0[t], 0, 0)),
        ],
        out_specs=pl.BlockSpec((BT, D), lambda t, s0, s1: (t, 0)),
    )
    return pl.pallas_call(
        _group_body,
        grid_spec=grid_spec,
        out_shape=jax.ShapeDtypeStruct((PADT, D), jnp.float32),
    )(tile_e, nreal, Xs, We_gate, We_up, We_down)


def _shared_body(x_ref, wsg_ref, wsu_ref, wsd_ref, wgate_ref, out_ref):
    x = x_ref[...]
    g = lax.dot_general(x, wsg_ref[...], (((1,), (1,)), ((), ())),
                        preferred_element_type=jnp.float32)
    u = lax.dot_general(x, wsu_ref[...], (((1,), (1,)), ((), ())),
                        preferred_element_type=jnp.float32)
    a = g * jax.nn.sigmoid(g) * u
    sh = lax.dot_general(a, wsd_ref[...], (((1,), (1,)), ((), ())),
                         preferred_element_type=jnp.float32)
    gate = jax.nn.sigmoid(
        lax.dot_general(x, wgate_ref[...], (((1,), (1,)), ((), ())),
                        preferred_element_type=jnp.float32))
    out_ref[...] = gate * sh


def _shared(x, Ws_gate, Ws_up, Ws_down, Wsg, bt=256):
    return pl.pallas_call(
        _shared_body,
        grid=(T // bt,),
        in_specs=[
            pl.BlockSpec((bt, D), lambda t: (t, 0)),
            pl.BlockSpec((SFF, D), lambda t: (0, 0)),
            pl.BlockSpec((SFF, D), lambda t: (0, 0)),
            pl.BlockSpec((D, SFF), lambda t: (0, 0)),
            pl.BlockSpec((1, D), lambda t: (0, 0)),
        ],
        out_specs=pl.BlockSpec((bt, D), lambda t: (t, 0)),
        out_shape=jax.ShapeDtypeStruct((T, D), jnp.float32),
    )(x, Ws_gate, Ws_up, Ws_down, Wsg)


def _combine_sc(hs, gs, pos_flat, wexp):
    mesh = plsc.VectorSubcoreMesh(core_axis_name="c", subcore_axis_name="s")
    tok_per_w = T // NW
    ch = 16
    nch = tok_per_w // ch

    @functools.partial(
        pl.kernel, mesh=mesh,
        out_type=jax.ShapeDtypeStruct((T, D), jnp.float32),
        scratch_types=[
            pltpu.VMEM((ch,), jnp.int32), pltpu.VMEM((ch,), jnp.int32),
            pltpu.VMEM((ch,), jnp.int32), pltpu.VMEM((ch,), jnp.int32),
            pltpu.VMEM((ch, K * 16), jnp.float32), pltpu.VMEM((ch, K * 16), jnp.float32),
            pltpu.VMEM((ch, D), jnp.float32), pltpu.VMEM((ch, D), jnp.float32),
            pltpu.VMEM((ch, D), jnp.float32), pltpu.VMEM((ch, D), jnp.float32),
            pltpu.VMEM((ch, D), jnp.float32), pltpu.VMEM((ch, D), jnp.float32),
            pltpu.SemaphoreType.DMA, pltpu.SemaphoreType.DMA,
            pltpu.SemaphoreType.DMA, pltpu.SemaphoreType.DMA,
        ],
    )
    def k(hs_hbm, gs_hbm, pf_hbm, we_hbm, out_hbm,
          i0a, i0b, i1a, i1b, wea, web, r0a, r0b, r1a, r1b, ga, gb,
          ls0, ls1, gs0, gs1):
        wid = lax.axis_index("s") * NC + lax.axis_index("c")
        i0_v = (i0a, i0b)
        i1_v = (i1a, i1b)
        we_v = (wea, web)
        r0_v = (r0a, r0b)
        r1_v = (r1a, r1b)
        g_v = (ga, gb)
        lsem = (ls0, ls1)
        gsem = (gs0, gs1)
        lh = {}
        gh = {}

        def fire_loads(c):
            b = c % 2
            base = wid * tok_per_w + c * ch
            lh[c] = (
                pltpu.async_copy(pf_hbm.at[pl.ds(base, ch)], i0_v[b], lsem[b]),
                pltpu.async_copy(pf_hbm.at[pl.ds(T + base, ch)], i1_v[b], lsem[b]),
                pltpu.async_copy(we_hbm.at[pl.ds(base, ch)], we_v[b], lsem[b]),
                pltpu.async_copy(gs_hbm.at[pl.ds(base, ch)], g_v[b], lsem[b]),
            )

        def fire_gathers(c):
            b = c % 2
            gh[c] = (
                pltpu.async_copy(hs_hbm.at[i0_v[b]], r0_v[b], gsem[b]),
                pltpu.async_copy(hs_hbm.at[i1_v[b]], r1_v[b], gsem[b]),
            )

        def add_and_store(c):
            b = c % 2
            base = wid * tok_per_w + c * ch

            def rbody(r, _):
                w0 = we_v[b][r, pl.ds(0, 16)]
                w1 = we_v[b][r, pl.ds(16, 16)]

                def jbody(j, _):
                    off = j * 16
                    r0_v[b][r, pl.ds(off, 16)] = (w0 * r0_v[b][r, pl.ds(off, 16)]
                                                  + w1 * r1_v[b][r, pl.ds(off, 16)]
                                                  + g_v[b][r, pl.ds(off, 16)])
                    return 0
                lax.fori_loop(0, D // 16, jbody, 0)
                return 0

            lax.fori_loop(0, ch, rbody, 0)
            pltpu.sync_copy(r0_v[b], out_hbm.at[pl.ds(base, ch)])

        fire_loads(0)
        fire_loads(1)
        for c in range(nch):
            for h in lh[c]:
                h.wait()
            fire_gathers(c)
            if c >= 1:
                for h in gh[c - 1]:
                    h.wait()
                add_and_store(c - 1)
                if c + 1 < nch:
                    fire_loads(c + 1)
        for h in gh[nch - 1]:
            h.wait()
        add_and_store(nch - 1)

    return k(hs, gs, pos_flat, wexp)


def kernel(hidden_states, Wg, We_gate, We_up, We_down, Ws_gate, Ws_up, Ws_down, Wsg):
    x = hidden_states.reshape(-1, hidden_states.shape[-1])
    pos_flat, wexp, tile_e, nreal = _router(x, Wg)
    Xs = _scatter_sc(x, pos_flat)
    gs = _shared(x, Ws_gate, Ws_up, Ws_down, Wsg)
    hs = _group_mm(tile_e, nreal, Xs, We_gate, We_up, We_down)
    out = _combine_sc(hs, gs, pos_flat, wexp)
    return out.reshape(hidden_states.shape)


# speedup vs baseline: 1.0939x; 1.0939x over previous
import functools

import jax
import jax.numpy as jnp
from jax import lax
from jax.experimental import pallas as pl
from jax.experimental.pallas import tpu as pltpu
from jax.experimental.pallas import tpu_sc as plsc

T = 2048
D = 1024
E = 8
K = 2
FF = 768
SFF = 2048

BT = 256
NT = 24
PADT = NT * BT

NC = 2
NS = 16
NW = NC * NS


def _router_body(x_ref, wg_ref, pos_ref, wexp_ref, te_ref, nr_ref):
    x = x_ref[...]
    logits = lax.dot_general(wg_ref[...], x, (((1,), (1,)), ((), ())),
                             preferred_element_type=jnp.float32)
    m = jnp.max(logits, axis=0, keepdims=True)
    ex = jnp.exp(logits - m)
    probs = ex / jnp.sum(ex, axis=0, keepdims=True)
    eids = lax.broadcasted_iota(jnp.int32, (E, T), 0)
    m1 = jnp.max(probs, axis=0, keepdims=True)
    i1 = jnp.min(jnp.where(probs == m1, eids, E), axis=0, keepdims=True)
    mask1 = eids == i1
    probs2 = jnp.where(mask1, -1.0, probs)
    m2 = jnp.max(probs2, axis=0, keepdims=True)
    i2 = jnp.min(jnp.where(probs2 == m2, eids, E), axis=0, keepdims=True)
    mask2 = eids == i2
    s = m1 + m2
    w1 = m1 / s
    w2 = m2 / s

    A = mask1.astype(jnp.float32) + mask2.astype(jnp.float32)
    r_io = lax.broadcasted_iota(jnp.int32, (128, 128), 0)
    c_io = lax.broadcasted_iota(jnp.int32, (128, 128), 1)
    LT = (r_io < c_io).astype(jnp.float32)
    carry = jnp.zeros((E, 1), jnp.float32)
    chunks = []
    for c in range(T // 128):
        Xc = A[:, c * 128:(c + 1) * 128]
        Sc = lax.dot_general(Xc, LT, (((1,), (0,)), ((), ())),
                             preferred_element_type=jnp.float32) + carry
        carry = carry + jnp.sum(Xc, axis=1, keepdims=True)
        chunks.append(Sc)
    S = jnp.concatenate(chunks, axis=1)
    counts = carry
    padded = jnp.ceil(counts / BT) * BT
    tr = lax.broadcasted_iota(jnp.int32, (E, E), 0)
    tc_ = lax.broadcasted_iota(jnp.int32, (E, E), 1)
    tri = (tc_ < tr).astype(jnp.float32)
    base = lax.dot_general(tri, padded, (((1,), (0,)), ((), ())),
                           preferred_element_type=jnp.float32)
    ptotal = jnp.sum(padded)

    P = base + S
    pos0 = jnp.sum(jnp.where(mask1, P, 0.0), axis=0, keepdims=True)
    pos1 = jnp.sum(jnp.where(mask2, P, 0.0), axis=0, keepdims=True)
    pos_ref[...] = jnp.concatenate([pos0, pos1], axis=1).astype(jnp.int32).reshape(K * T)
    w1t = jnp.transpose(w1, (1, 0))
    w2t = jnp.transpose(w2, (1, 0))
    wexp_ref[...] = jnp.concatenate(
        [jnp.broadcast_to(w1t, (T, 16)), jnp.broadcast_to(w2t, (T, 16))], axis=1)

    tl = lax.broadcasted_iota(jnp.int32, (1, 128), 1).astype(jnp.float32) * BT
    belongs = (tl >= base) & (tl < base + padded)
    eids_p = lax.broadcasted_iota(jnp.int32, (E, 128), 0)
    tile_e = jnp.sum(jnp.where(belongs, eids_p, 0), axis=0, keepdims=True)
    tile_e = jnp.where(tl < ptotal, tile_e, E - 1)
    te_ref[...] = tile_e.reshape(128)
    nreal = (ptotal / BT).astype(jnp.int32)
    nr_ref[...] = jnp.zeros((8,), jnp.int32) + nreal


def _router(x, Wg):
    return pl.pallas_call(
        _router_body,
        out_shape=(
            jax.ShapeDtypeStruct((K * T,), jnp.int32),
            jax.ShapeDtypeStruct((T, K * 16), jnp.float32),
            jax.ShapeDtypeStruct((128,), jnp.int32),
            jax.ShapeDtypeStruct((8,), jnp.int32),
        ),
    )(x, Wg)


def _scatter_sc(x, pos_flat):
    mesh = plsc.VectorSubcoreMesh(core_axis_name="c", subcore_axis_name="s")
    tok_per_w = T // NW
    ch = 32
    nch = tok_per_w // ch

    @functools.partial(
        pl.kernel, mesh=mesh,
        out_type=jax.ShapeDtypeStruct((PADT, D), jnp.float32),
        scratch_types=[
            pltpu.VMEM((ch,), jnp.int32), pltpu.VMEM((ch,), jnp.int32),
            pltpu.VMEM((ch,), jnp.int32), pltpu.VMEM((ch,), jnp.int32),
            pltpu.VMEM((ch, D), jnp.float32), pltpu.VMEM((ch, D), jnp.float32),
            pltpu.SemaphoreType.DMA, pltpu.SemaphoreType.DMA,
            pltpu.SemaphoreType.DMA, pltpu.SemaphoreType.DMA,
        ],
    )
    def k(x_hbm, pos_hbm, xs_out,
          i0a, i0b, i1a, i1b, rows0, rows1, ls0, ls1, ss0, ss1):
        wid = lax.axis_index("s") * NC + lax.axis_index("c")
        i0_v = (i0a, i0b)
        i1_v = (i1a, i1b)
        rows_v = (rows0, rows1)
        lsem = (ls0, ls1)
        ssem = (ss0, ss1)
        lh = {}
        sh = {}

        def fire_loads(c):
            b = c % 2
            base = wid * tok_per_w + c * ch
            lh[c] = (
                pltpu.async_copy(pos_hbm.at[pl.ds(base, ch)], i0_v[b], lsem[b]),
                pltpu.async_copy(pos_hbm.at[pl.ds(T + base, ch)], i1_v[b], lsem[b]),
                pltpu.async_copy(x_hbm.at[pl.ds(base, ch)], rows_v[b], lsem[b]),
            )

        for c in range(nch):
            fire_loads(c)
        for c in range(nch):
            b = c % 2
            for h in lh[c]:
                h.wait()
            sh[c] = (
                pltpu.async_copy(rows_v[b], xs_out.at[i0_v[b]], ssem[b]),
                pltpu.async_copy(rows_v[b], xs_out.at[i1_v[b]], ssem[b]),
            )
        for c in range(nch):
            for h in sh[c]:
                h.wait()

    return k(x, pos_flat)


def _group_body(s0_ref, s1_ref, xs_ref, wg_ref, wu_ref, wd_ref, out_ref):
    t = pl.program_id(0)

    @pl.when(t < s1_ref[0])
    def _():
        x = xs_ref[...]
        g = lax.dot_general(x, wg_ref[0], (((1,), (1,)), ((), ())),
                            preferred_element_type=jnp.float32)
        u = lax.dot_general(x, wu_ref[0], (((1,), (1,)), ((), ())),
                            preferred_element_type=jnp.float32)
        a = g * jax.nn.sigmoid(g) * u
        out_ref[...] = lax.dot_general(a, wd_ref[0], (((1,), (1,)), ((), ())),
                                       preferred_element_type=jnp.float32)


def _group_mm(tile_e, nreal, Xs, We_gate, We_up, We_down):
    grid_spec = pltpu.PrefetchScalarGridSpec(
        num_scalar_prefetch=2,
        grid=(NT,),
        in_specs=[
            pl.BlockSpec((BT, D), lambda t, s0, s1: (t, 0)),
            pl.BlockSpec((1, FF, D), lambda t, s0, s1: (s0[t], 0, 0)),
            pl.BlockSpec((1, FF, D), lambda t, s0, s1: (s0[t], 0, 0)),
            pl.BlockSpec((1, D, FF), lambda t, s0, s1: (s0[t], 0, 0)),
        ],
        out_specs=pl.BlockSpec((BT, D), lambda t, s0, s1: (t, 0)),
    )
    return pl.pallas_call(
        _group_body,
        grid_spec=grid_spec,
        out_shape=jax.ShapeDtypeStruct((PADT, D), jnp.float32),
    )(tile_e, nreal, Xs, We_gate, We_up, We_down)


def _shared_body(x_ref, wsg_ref, wsu_ref, wsd_ref, wgate_ref, out_ref):
    x = x_ref[...]
    g = lax.dot_general(x, wsg_ref[...], (((1,), (1,)), ((), ())),
                        preferred_element_type=jnp.float32)
    u = lax.dot_general(x, wsu_ref[...], (((1,), (1,)), ((), ())),
                        preferred_element_type=jnp.float32)
    a = g * jax.nn.sigmoid(g) * u
    sh = lax.dot_general(a, wsd_ref[...], (((1,), (1,)), ((), ())),
                         preferred_element_type=jnp.float32)
    gate = jax.nn.sigmoid(
        lax.dot_general(x, wgate_ref[...], (((1,), (1,)), ((), ())),
                        preferred_element_type=jnp.float32))
    out_ref[...] = gate * sh


def _shared(x, Ws_gate, Ws_up, Ws_down, Wsg, bt=256):
    return pl.pallas_call(
        _shared_body,
        grid=(T // bt,),
        in_specs=[
            pl.BlockSpec((bt, D), lambda t: (t, 0)),
            pl.BlockSpec((SFF, D), lambda t: (0, 0)),
            pl.BlockSpec((SFF, D), lambda t: (0, 0)),
            pl.BlockSpec((D, SFF), lambda t: (0, 0)),
            pl.BlockSpec((1, D), lambda t: (0, 0)),
        ],
        out_specs=pl.BlockSpec((bt, D), lambda t: (t, 0)),
        out_shape=jax.ShapeDtypeStruct((T, D), jnp.float32),
    )(x, Ws_gate, Ws_up, Ws_down, Wsg)


def _combine_sc(hs, gs, pos_flat, wexp):
    mesh = plsc.VectorSubcoreMesh(core_axis_name="c", subcore_axis_name="s")
    tok_per_w = T // NW
    ch = 16
    nch = tok_per_w // ch

    @functools.partial(
        pl.kernel, mesh=mesh,
        out_type=jax.ShapeDtypeStruct((T, D), jnp.float32),
        scratch_types=[
            pltpu.VMEM((ch,), jnp.int32), pltpu.VMEM((ch,), jnp.int32),
            pltpu.VMEM((ch,), jnp.int32), pltpu.VMEM((ch,), jnp.int32),
            pltpu.VMEM((ch, K * 16), jnp.float32), pltpu.VMEM((ch, K * 16), jnp.float32),
            pltpu.VMEM((ch, D), jnp.float32), pltpu.VMEM((ch, D), jnp.float32),
            pltpu.VMEM((ch, D), jnp.float32), pltpu.VMEM((ch, D), jnp.float32),
            pltpu.VMEM((ch, D), jnp.float32), pltpu.VMEM((ch, D), jnp.float32),
            pltpu.SemaphoreType.DMA, pltpu.SemaphoreType.DMA,
            pltpu.SemaphoreType.DMA, pltpu.SemaphoreType.DMA,
        ],
    )
    def k(hs_hbm, gs_hbm, pf_hbm, we_hbm, out_hbm,
          i0a, i0b, i1a, i1b, wea, web, r0a, r0b, r1a, r1b, ga, gb,
          ls0, ls1, gs0, gs1):
        wid = lax.axis_index("s") * NC + lax.axis_index("c")
        i0_v = (i0a, i0b)
        i1_v = (i1a, i1b)
        we_v = (wea, web)
        r0_v = (r0a, r0b)
        r1_v = (r1a, r1b)
        g_v = (ga, gb)
        lsem = (ls0, ls1)
        gsem = (gs0, gs1)
        lh = {}
        gh = {}

        def fire_loads(c):
            b = c % 2
            base = wid * tok_per_w + c * ch
            lh[c] = (
                pltpu.async_copy(pf_hbm.at[pl.ds(base, ch)], i0_v[b], lsem[b]),
                pltpu.async_copy(pf_hbm.at[pl.ds(T + base, ch)], i1_v[b], lsem[b]),
                pltpu.async_copy(we_hbm.at[pl.ds(base, ch)], we_v[b], lsem[b]),
                pltpu.async_copy(gs_hbm.at[pl.ds(base, ch)], g_v[b], lsem[b]),
            )

        def fire_gathers(c):
            b = c % 2
            gh[c] = (
                pltpu.async_copy(hs_hbm.at[i0_v[b]], r0_v[b], gsem[b]),
                pltpu.async_copy(hs_hbm.at[i1_v[b]], r1_v[b], gsem[b]),
            )

        def add_and_store(c):
            b = c % 2
            base = wid * tok_per_w + c * ch

            def rbody(r, _):
                w0 = we_v[b][r, pl.ds(0, 16)]
                w1 = we_v[b][r, pl.ds(16, 16)]
                for j in range(D // 16):
                    off = j * 16
                    r0_v[b][r, pl.ds(off, 16)] = (w0 * r0_v[b][r, pl.ds(off, 16)]
                                                  + w1 * r1_v[b][r, pl.ds(off, 16)]
                                                  + g_v[b][r, pl.ds(off, 16)])
                return 0

            lax.fori_loop(0, ch, rbody, 0)
            pltpu.sync_copy(r0_v[b], out_hbm.at[pl.ds(base, ch)])

        fire_loads(0)
        fire_loads(1)
        for c in range(nch):
            for h in lh[c]:
                h.wait()
            fire_gathers(c)
            if c >= 1:
                for h in gh[c - 1]:
                    h.wait()
                add_and_store(c - 1)
                if c + 1 < nch:
                    fire_loads(c + 1)
        for h in gh[nch - 1]:
            h.wait()
        add_and_store(nch - 1)

    return k(hs, gs, pos_flat, wexp)


def kernel(hidden_states, Wg, We_gate, We_up, We_down, Ws_gate, Ws_up, Ws_down, Wsg):
    x = hidden_states.reshape(-1, hidden_states.shape[-1])
    pos_flat, wexp, tile_e, nreal = _router(x, Wg)
    Xs = _scatter_sc(x, pos_flat)
    gs = _shared(x, Ws_gate, Ws_up, Ws_down, Wsg)
    hs = _group_mm(tile_e, nreal, Xs, We_gate, We_up, We_down)
    out = _combine_sc(hs, gs, pos_flat, wexp)
    return out.reshape(hidden_states.shape)
